# Initial kernel scaffold; baseline (speedup 1.0000x reference)
import jax
import jax.numpy as jnp
from jax.experimental import pallas as pl


def kernel(x, edge_index, W_pe, b_pe, W_ce, b_ce, W_m1, b_m1, W_m2, b_m2, W_rel, b_rel, W_root):
    raise NotImplementedError("write your pallas kernel here")


# speedup vs baseline: 2.4388x; 2.4388x over previous
import functools

import jax
import jax.numpy as jnp
from jax import lax
from jax.experimental import pallas as pl
from jax.experimental.pallas import tpu as pltpu
from jax.experimental.pallas import tpu_sc as plsc

N = 10000
E = 320000
D = 128

NC = 2
NS = 16
NW = NC * NS
PER_TILE = E // NW
CS = 80
NCH = PER_TILE // CS
N_PAD = 10240
NROWS_T = N_PAD // NS
ZB = 128

_mesh = plsc.VectorSubcoreMesh(
    core_axis_name="c", subcore_axis_name="s", num_cores=NC, num_subcores=NS)


@functools.partial(
    pl.kernel,
    out_type=[jax.ShapeDtypeStruct((E, D), jnp.float32),
              jax.ShapeDtypeStruct((E, D), jnp.float32)],
    mesh=_mesh,
    scratch_types=[
        pltpu.VMEM((PER_TILE,), jnp.int32),
        pltpu.VMEM((PER_TILE,), jnp.int32),
        pltpu.VMEM((CS, D), jnp.float32),
        pltpu.VMEM((CS, D), jnp.float32),
    ],
)
def _sc_gather(x_hbm, src_hbm, dst_hbm, gs_hbm, gd_hbm,
               idx_s, idx_d, buf_s, buf_d):
    cid = lax.axis_index("c")
    sid = lax.axis_index("s")
    wid = cid * NS + sid
    base = wid * PER_TILE
    pltpu.sync_copy(src_hbm.at[pl.ds(base, PER_TILE)], idx_s)
    pltpu.sync_copy(dst_hbm.at[pl.ds(base, PER_TILE)], idx_d)

    def body(j, carry):
        off = j * CS
        pltpu.sync_copy(x_hbm.at[idx_s.at[pl.ds(off, CS)]], buf_s)
        pltpu.sync_copy(buf_s, gs_hbm.at[pl.ds(base + off, CS)])
        pltpu.sync_copy(x_hbm.at[idx_d.at[pl.ds(off, CS)]], buf_d)
        pltpu.sync_copy(buf_d, gd_hbm.at[pl.ds(base + off, CS)])
        return carry

    lax.fori_loop(0, NCH, body, 0)


@functools.partial(
    pl.kernel,
    out_type=jax.ShapeDtypeStruct((NC, N_PAD, D), jnp.float32),
    mesh=_mesh,
    scratch_types=[
        pltpu.VMEM((NCH, CS), jnp.int32),
        pltpu.VMEM((CS, D), jnp.float32),
        pltpu.VMEM((ZB, D), jnp.float32),
        pltpu.VMEM_SHARED((N_PAD, D), jnp.float32),
    ],
)
def _sc_scatter(msg_hbm, dst3_hbm, out_hbm, idx_all, rows, zbuf, agg_sh):
    cid = lax.axis_index("c")
    sid = lax.axis_index("s")
    wid = cid * NS + sid

    def zb(t, carry):
        i = t // (D // 16)
        k = t % (D // 16)
        zbuf[i, pl.ds(k * 16, 16)] = jnp.zeros((16,), jnp.float32)
        return carry

    lax.fori_loop(0, ZB * (D // 16), zb, 0)
    row0 = sid * NROWS_T
    for m in range(NROWS_T // ZB):
        pltpu.sync_copy(zbuf, agg_sh.at[pl.ds(row0 + m * ZB, ZB)])
    plsc.subcore_barrier()

    ebase = wid * PER_TILE
    pltpu.sync_copy(dst3_hbm.at[wid], idx_all)

    def body(j, carry):
        pltpu.sync_copy(msg_hbm.at[pl.ds(ebase + j * CS, CS)], rows)
        pltpu.sync_copy(rows, agg_sh.at[idx_all.at[j]], add=True)
        return carry

    lax.fori_loop(0, NCH, body, 0)
    plsc.subcore_barrier()

    for m in range(NROWS_T // ZB):
        r = row0 + m * ZB
        pltpu.sync_copy(agg_sh.at[pl.ds(r, ZB)], zbuf)
        pltpu.sync_copy(zbuf, out_hbm.at[cid].at[pl.ds(r, ZB)])


BE = 2560


def _edge_body(gs, gd, wpe, wce, m1p, m1c, m1d, bpe, bce, bm1, w2, bm2, msg):
    xs = gs[...]
    xd = gd[...]
    t1 = jnp.dot(xs, wpe[...], preferred_element_type=jnp.float32) + bpe[...]
    t1 = jnp.where(t1 >= 0, t1, 0.01 * t1)
    t2 = jnp.dot(xd, wce[...], preferred_element_type=jnp.float32) + bce[...]
    t2 = jnp.where(t2 >= 0, t2, 0.01 * t2)
    pre = (jnp.dot(t1, m1p[...], preferred_element_type=jnp.float32)
           + jnp.dot(t2, m1c[...], preferred_element_type=jnp.float32)
           + jnp.dot(jnp.abs(xs - xd), m1d[...],
                     preferred_element_type=jnp.float32)
           + bm1[...])
    h = jnp.maximum(pre, 0.0)
    z = jnp.sum(h * w2[...], axis=1, keepdims=True) + bm2[...]
    ew = 1.0 / (1.0 + jnp.exp(-z))
    msg[...] = xs * ew


def _edge_mlp(gs, gd, wpe, wce, m1p, m1c, m1d, bpe, bce, bm1, w2, bm2):
    full = lambda shp: pl.BlockSpec(shp, lambda i: (0,) * len(shp))
    return pl.pallas_call(
        _edge_body,
        grid=(E // BE,),
        in_specs=[
            pl.BlockSpec((BE, D), lambda i: (i, 0)),
            pl.BlockSpec((BE, D), lambda i: (i, 0)),
            full((D, D)), full((D, D)), full((D, D)), full((D, D)),
            full((D, D)),
            full((1, D)), full((1, D)), full((1, D)), full((1, D)),
            full((1, 1)),
        ],
        out_specs=pl.BlockSpec((BE, D), lambda i: (i, 0)),
        out_shape=jax.ShapeDtypeStruct((E, D), jnp.float32),
    )(gs, gd, wpe, wce, m1p, m1c, m1d, bpe, bce, bm1, w2, bm2)


BN = 2000


def _post_body(a0, a1, xb, wrel, wroot, brel, out):
    agg = a0[...] + a1[...]
    out[...] = (jnp.dot(agg, wrel[...], preferred_element_type=jnp.float32)
                + jnp.dot(xb[...], wroot[...],
                          preferred_element_type=jnp.float32)
                + brel[...])


def _post(a0, a1, x, wrel, wroot, brel):
    full = lambda shp: pl.BlockSpec(shp, lambda i: (0,) * len(shp))
    return pl.pallas_call(
        _post_body,
        grid=(N // BN,),
        in_specs=[
            pl.BlockSpec((BN, D), lambda i: (i, 0)),
            pl.BlockSpec((BN, D), lambda i: (i, 0)),
            pl.BlockSpec((BN, D), lambda i: (i, 0)),
            full((D, D)), full((D, D)), full((1, D)),
        ],
        out_specs=pl.BlockSpec((BN, D), lambda i: (i, 0)),
        out_shape=jax.ShapeDtypeStruct((N, D), jnp.float32),
    )(a0, a1, x, wrel, wroot, brel)


def kernel(x, edge_index, W_pe, b_pe, W_ce, b_ce, W_m1, b_m1, W_m2, b_m2,
           W_rel, b_rel, W_root):
    src = edge_index[0]
    dst = edge_index[1]
    dst3 = dst.reshape(NW, NCH, CS)

    gs, gd = _sc_gather(x, src, dst)

    msg = _edge_mlp(
        gs, gd, W_pe, W_ce,
        W_m1[:D], W_m1[D:2 * D], W_m1[2 * D:],
        b_pe.reshape(1, D), b_ce.reshape(1, D), b_m1.reshape(1, D),
        W_m2.reshape(1, D), b_m2.reshape(1, 1))

    aggs = _sc_scatter(msg, dst3)

    return _post(aggs[0], aggs[1], x, W_rel, W_root, b_rel.reshape(1, D))
